# Initial kernel scaffold; baseline (speedup 1.0000x reference)
import jax
import jax.numpy as jnp
from jax.experimental import pallas as pl


def kernel(y_true, y_pred, B):
    raise NotImplementedError("write your pallas kernel here")


# speedup vs baseline: 5.6607x; 5.6607x over previous
import functools

import jax
import jax.numpy as jnp
from jax import lax
from jax.experimental import pallas as pl
from jax.experimental.pallas import tpu as pltpu
from jax.experimental.pallas import tpu_sc as plsc

_NC = 2
_NS = 16
_NW = _NC * _NS
_L = 16

_V = 24
_CHUNK = 2048
_NACC = 8


def _sc_partials(y_flat, p_flat, b_flat):
    n_tok = y_flat.shape[0]
    tok_per_w = n_tok // _NW
    n_chunks = tok_per_w // _CHUNK
    groups = _CHUNK // _L

    mesh = plsc.VectorSubcoreMesh(core_axis_name="c", subcore_axis_name="s")

    @functools.partial(
        pl.kernel,
        out_type=jax.ShapeDtypeStruct((_NW, _L), jnp.float32),
        mesh=mesh,
        scratch_types=[
            pltpu.VMEM((_CHUNK,), jnp.int32),
            pltpu.VMEM((_CHUNK * _V,), jnp.float32),
            pltpu.VMEM((_V * _V,), jnp.float32),
            pltpu.VMEM((_L,), jnp.float32),
            pltpu.SemaphoreType.DMA,
        ],
        compiler_params=pltpu.CompilerParams(needs_layout_passes=False),
    )
    def sc_fn(y_hbm, p_hbm, b_hbm, out_hbm, y_buf, p_buf, b_vmem, acc_vmem, sem):
        wid = lax.axis_index("s") * _NC + lax.axis_index("c")
        wbase = wid * tok_per_w

        pltpu.sync_copy(b_hbm, b_vmem)
        col_iota = lax.iota(jnp.int32, _L) * _V

        def chunk_body(ci, accs):
            tbase = wbase + ci * _CHUNK
            pltpu.sync_copy(y_hbm.at[pl.ds(tbase, _CHUNK)], y_buf)
            pltpu.sync_copy(p_hbm.at[pl.ds(tbase * _V, _CHUNK * _V)], p_buf)

            def group_body(g, accs):
                y_v = y_buf[pl.ds(g * _L, _L)]
                rowoff = y_v * _V
                pbase = g * (_L * _V)
                accs = list(accs)
                for k in range(_V):
                    pcol = plsc.load_gather(p_buf, [col_iota + (pbase + k)])
                    bval = plsc.load_gather(b_vmem, [rowoff + k])
                    accs[k % _NACC] = accs[k % _NACC] + pcol * bval
                return tuple(accs)

            return lax.fori_loop(0, groups, group_body, accs)

        zero = jnp.zeros((_L,), jnp.float32)
        accs = lax.fori_loop(0, n_chunks, chunk_body, (zero,) * _NACC)

        total = accs[0]
        for a in accs[1:]:
            total = total + a
        acc_vmem[...] = total
        pltpu.sync_copy(acc_vmem, out_hbm.at[wid])

    return sc_fn(y_flat, p_flat, b_flat)


def kernel(y_true, y_pred, B):
    y_flat = y_true.reshape(-1)
    p_flat = y_pred.reshape(-1)
    b_flat = B.reshape(-1)
    partials = _sc_partials(y_flat, p_flat, b_flat)
    return jnp.sum(partials)
